# baseline (speedup 1.0000x reference)
import functools

import jax
import jax.numpy as jnp
from jax import lax
from jax.experimental import pallas as pl
from jax.experimental.pallas import tpu as pltpu
from jax.experimental.pallas import tpu_sc as plsc

GROUP = 128
NLEV = 16
LANES = 16
VECS = GROUP // LANES


@functools.cache
def _make_sc_quantize(n_rows: int, n_cols: int):
  info = plsc.get_sparse_core_info()
  nw = info.num_cores * info.num_subcores
  gpr = n_cols // GROUP
  rows_per_w = n_rows // nw
  slab = 8
  n_chunks = rows_per_w // slab
  slab_groups = slab * gpr
  mesh = plsc.VectorSubcoreMesh(core_axis_name="c", subcore_axis_name="s")

  @functools.partial(
      pl.kernel,
      out_type=jax.ShapeDtypeStruct((n_rows, n_cols), jnp.float32),
      mesh=mesh,
      scratch_types=[
          pltpu.VMEM((slab, n_cols), jnp.float32),
          pltpu.VMEM((slab_groups, NLEV), jnp.float32),
          pltpu.VMEM((slab_groups, NLEV), jnp.float32),
          pltpu.VMEM((slab, n_cols), jnp.float32),
      ],
  )
  def body(x_hbm, lv_hbm, bd_hbm, out_hbm, x_v, l_v, b_v, o_v):
    wid = lax.axis_index("s") * info.num_cores + lax.axis_index("c")
    row_base = wid * rows_per_w

    def do_chunk(ci, carry):
      r0 = row_base + ci * slab
      g0 = r0 * gpr
      pltpu.sync_copy(x_hbm.at[pl.ds(r0, slab)], x_v)
      pltpu.sync_copy(lv_hbm.at[pl.ds(g0, slab_groups)], l_v)
      pltpu.sync_copy(bd_hbm.at[pl.ds(g0, slab_groups)], b_v)

      def do_row(s, carry2):
        def do_group(j, carry3):
          g = s * gpr + j
          bvec = b_v[g, :]
          lvec = l_v[g, :]
          for v in range(VECS):
            xv = x_v[s, pl.ds(j * GROUP + v * LANES, LANES)]
            idx = jnp.zeros((LANES,), jnp.int32)
            for w, off in ((8, 7), (4, 3), (2, 1), (1, 0)):
              probe = jnp.take_along_axis(
                  bvec, idx + off, axis=0, mode="promise_in_bounds")
              idx = jnp.where(xv > probe, idx + w, idx)
            xq = jnp.take_along_axis(lvec, idx, axis=0,
                                     mode="promise_in_bounds")
            o_v[s, pl.ds(j * GROUP + v * LANES, LANES)] = (xq - xv) + xv
          return carry3

        return lax.fori_loop(0, gpr, do_group, carry2)

      lax.fori_loop(0, slab, do_row, 0)
      pltpu.sync_copy(o_v, out_hbm.at[pl.ds(r0, slab)])
      return carry

    lax.fori_loop(0, n_chunks, do_chunk, 0)

  return body


def kernel(x, levels, borders):
  bd = jnp.concatenate([borders, borders[:, -1:]], axis=1)
  return _make_sc_quantize(*x.shape)(x, levels, bd)
